# baseline (speedup 1.0000x reference)
import dataclasses
import functools

import jax
import jax.numpy as jnp
from jax import lax
from jax.experimental import pallas as pl
from jax.experimental.pallas import tpu as pltpu
from jax.experimental.pallas import tpu_sc as plsc

N = 1024
E = 32768
SDIM = 256
VDIM = 64
EDGE_DIM = 16
NUM_GRAPHS = 32

NKEYS = N * N
NW = 32
RANGE = NKEYS // NW
EW = E // NW
CHUNK = 128
KEY_CHUNK = 2048
NSENT = 128

_sc_mesh = plsc.VectorSubcoreMesh(core_axis_name="c", subcore_axis_name="s")


def _sc_params(tc_tiling):
    cp = pltpu.CompilerParams()
    fields = pltpu.CompilerParams.__dataclass_fields__
    if "needs_layout_passes" in fields:
        cp = dataclasses.replace(cp, needs_layout_passes=False)
    if "use_tc_tiling_on_sc" in fields:
        cp = dataclasses.replace(cp, use_tc_tiling_on_sc=tc_tiling)
    return cp


def _node_body(s_ref, v0_ref, v1_ref, v2_ref, p16_ref, b_ref, Ws_ref, bs_ref,
               Wc_ref, Wa_ref, ba_ref, s1_ref, atoms_ref, c16_ref):
    f32 = jnp.float32
    s1 = jax.nn.silu(
        jnp.dot(s_ref[...], Ws_ref[...], preferred_element_type=f32)
        + bs_ref[...])
    s1_ref[...] = s1
    atoms_ref[...] = (
        jnp.dot(s1, Wa_ref[...], preferred_element_type=f32) + ba_ref[...])

    Wc = Wc_ref[...]
    c3 = jnp.concatenate(
        [jnp.dot(v0_ref[...], Wc, preferred_element_type=f32),
         jnp.dot(v1_ref[...], Wc, preferred_element_type=f32),
         jnp.dot(v2_ref[...], Wc, preferred_element_type=f32),
         jnp.zeros((N, 13), f32)], axis=1)
    c16 = c3 + p16_ref[...]
    iota = lax.broadcasted_iota(jnp.int32, (N, NUM_GRAPHS), 1)
    oh = (b_ref[...] == iota).astype(f32)
    ones = jnp.ones((N, 1), f32)
    seg = lax.dot_general(oh, c16, (((0,), (0,)), ((), ())),
                          preferred_element_type=f32)
    cnt = lax.dot_general(oh, ones, (((0,), (0,)), ((), ())),
                          preferred_element_type=f32)
    mean = seg / jnp.maximum(cnt, 1.0)
    c16_ref[...] = c16 - jnp.dot(oh, mean, preferred_element_type=f32)


def _node_kernel(s, v0, v1, v2, p16, batch2, W_shared, b_shared, W_coords,
                 W_atoms, b_atoms):
    f32 = jnp.float32
    return pl.pallas_call(
        _node_body,
        out_shape=(
            jax.ShapeDtypeStruct((N, SDIM), f32),
            jax.ShapeDtypeStruct((N, 16), f32),
            jax.ShapeDtypeStruct((N, 16), f32),
        ),
    )(s, v0, v1, v2, p16, batch2, W_shared, b_shared, W_coords, W_atoms,
      b_atoms)


def _table_body(key_hbm, neg1_hbm, table_hbm, tab, kb0, kb1, sem0, sem1):
    wid = lax.axis_index("s") * 2 + lax.axis_index("c")
    pltpu.sync_copy(neg1_hbm, tab)
    iota = lax.broadcasted_iota(jnp.int32, (16,), 0)

    def start_load(c0, kb, sem):
        pltpu.async_copy(key_hbm.at[pl.ds(c0, KEY_CHUNK)], kb, sem)

    def wait_load(kb, sem):
        pltpu.make_async_copy(key_hbm.at[pl.ds(0, KEY_CHUNK)], kb, sem).wait()

    def process(c0, kb):
        @pl.loop(0, KEY_CHUNK, step=128)
        def _vreg(t):
            for u in range(0, 128, 16):
                key = kb[pl.ds(t + u, 16)]
                inr = lax.shift_right_logical(key, 15) == wid
                _, keep = plsc.scan_count(key)
                m = jnp.logical_and(inr, keep)
                idx = jnp.bitwise_and(key, RANGE - 1)
                kvec = c0 + t + u + iota
                plsc.store_scatter(tab, [idx], kvec, mask=m)

    start_load(0, kb0, sem0)

    @pl.loop(0, E, step=2 * KEY_CHUNK)
    def _chunk(c0):
        wait_load(kb0, sem0)

        @pl.when(c0 + KEY_CHUNK < E)
        def _():
            start_load(c0 + KEY_CHUNK, kb1, sem1)

        process(c0, kb0)
        wait_load(kb1, sem1)

        @pl.when(c0 + 2 * KEY_CHUNK < E)
        def _():
            start_load(c0 + 2 * KEY_CHUNK, kb0, sem0)

        process(c0 + KEY_CHUNK, kb1)

    pltpu.sync_copy(tab, table_hbm.at[pl.ds(wid * RANGE, RANGE)])


def _build_table(key, neg1):
    i32 = jnp.int32
    kern = pl.kernel(
        _table_body,
        out_type=jax.ShapeDtypeStruct((NKEYS,), i32),
        mesh=_sc_mesh,
        scratch_types=[
            pltpu.VMEM((RANGE,), i32),
            pltpu.VMEM((KEY_CHUNK,), i32),
            pltpu.VMEM((KEY_CHUNK,), i32),
            pltpu.SemaphoreType.DMA,
            pltpu.SemaphoreType.DMA,
        ],
        compiler_params=_sc_params(False),
    )
    return kern(key, neg1)


def _esym_body(j_hbm, i_hbm, key_hbm, rkey_hbm, tab_hbm, e17_hbm, c16_hbm,
               esym_hbm, cdiff_hbm,
               jb, ib, keyb, rkeyb, fwdb, bwdb, ef, eb, cif, cjf,
               shr_c, csem, tsem, esem, ssem):
    cid = lax.axis_index("c")
    sid = lax.axis_index("s")
    wid = sid * 2 + cid
    e0 = wid * EW

    @pl.when(sid == 0)
    def _():
        pltpu.sync_copy(c16_hbm, shr_c)

    pltpu.sync_copy(j_hbm.at[pl.ds(e0, EW)], jb)
    pltpu.sync_copy(i_hbm.at[pl.ds(e0, EW)], ib)
    pltpu.sync_copy(key_hbm.at[pl.ds(e0, EW)], keyb)
    pltpu.sync_copy(rkey_hbm.at[pl.ds(e0, EW)], rkeyb)
    iota = lax.broadcasted_iota(jnp.int32, (16,), 0)
    plsc.subcore_barrier()

    ccps = []
    for c0 in range(0, EW, CHUNK):
        ccps.append(pltpu.async_copy(
            shr_c.at[ib.at[pl.ds(c0, CHUNK)]],
            cif.at[pl.ds(c0, CHUNK)], csem))
        ccps.append(pltpu.async_copy(
            shr_c.at[jb.at[pl.ds(c0, CHUNK)]],
            cjf.at[pl.ds(c0, CHUNK)], csem))

    tcps = []
    for c0 in range(0, EW, CHUNK):
        tcps.append(pltpu.async_copy(
            tab_hbm.at[keyb.at[pl.ds(c0, CHUNK)]],
            fwdb.at[pl.ds(c0, CHUNK)], tsem))
        tcps.append(pltpu.async_copy(
            tab_hbm.at[rkeyb.at[pl.ds(c0, CHUNK)]],
            bwdb.at[pl.ds(c0, CHUNK)], tsem))
    for cp in tcps:
        cp.wait()

    @pl.loop(0, EW, step=16)
    def _fix(t):
        b = bwdb[pl.ds(t, 16)]
        sent = E + jnp.bitwise_and(t + iota, NSENT - 1)
        bwdb[pl.ds(t, 16)] = jnp.where(b < 0, sent, b)

    ecps = []
    for c0 in range(0, EW, CHUNK):
        ecps.append(pltpu.async_copy(
            e17_hbm.at[fwdb.at[pl.ds(c0, CHUNK)]],
            ef.at[pl.ds(c0, CHUNK)], esem))
        ecps.append(pltpu.async_copy(
            e17_hbm.at[bwdb.at[pl.ds(c0, CHUNK)]],
            eb.at[pl.ds(c0, CHUNK)], esem))
    for cp in ecps:
        cp.wait()
    for cp in ccps:
        cp.wait()

    @pl.loop(0, EW, step=4)
    def _row(r):
        for u in range(4):
            ef[r + u] = (ef[r + u] + eb[r + u]) * 0.5
            cif[r + u] = cif[r + u] - cjf[r + u]

    pltpu.async_copy(ef, esym_hbm.at[pl.ds(e0, EW)], ssem)
    pltpu.async_copy(cif, cdiff_hbm.at[pl.ds(e0, EW)], ssem)
    pltpu.make_async_copy(ef, esym_hbm.at[pl.ds(e0, EW)], ssem).wait()
    pltpu.make_async_copy(cif, cdiff_hbm.at[pl.ds(e0, EW)], ssem).wait()


def _esym_kernel(j, i, key, rkey, table, e17, c16):
    f32 = jnp.float32
    i32 = jnp.int32
    kern = pl.kernel(
        _esym_body,
        out_type=(
            jax.ShapeDtypeStruct((E, EDGE_DIM), f32),
            jax.ShapeDtypeStruct((E, EDGE_DIM), f32),
        ),
        mesh=_sc_mesh,
        scratch_types=[
            pltpu.VMEM((EW,), i32),
            pltpu.VMEM((EW,), i32),
            pltpu.VMEM((EW,), i32),
            pltpu.VMEM((EW,), i32),
            pltpu.VMEM((EW,), i32),
            pltpu.VMEM((EW,), i32),
            pltpu.VMEM((EW, EDGE_DIM), f32),
            pltpu.VMEM((EW, EDGE_DIM), f32),
            pltpu.VMEM((EW, EDGE_DIM), f32),
            pltpu.VMEM((EW, EDGE_DIM), f32),
            pltpu.VMEM_SHARED((N, EDGE_DIM), f32),
            pltpu.SemaphoreType.DMA,
            pltpu.SemaphoreType.DMA,
            pltpu.SemaphoreType.DMA,
            pltpu.SemaphoreType.DMA,
        ],
        compiler_params=_sc_params(False),
    )
    return kern(j, i, key, rkey, table, e17, c16)


def _sgather_body(j_hbm, i_hbm, s1_hbm, s1i_hbm, s1j_hbm,
                  jb, ib, sa, sb, sa2, sb2, shr, gsem, ssem):
    cid = lax.axis_index("c")
    sid = lax.axis_index("s")
    wid = sid * 2 + cid
    e0 = wid * EW

    @pl.when(sid == 0)
    def _():
        pltpu.sync_copy(s1_hbm, shr)

    pltpu.sync_copy(j_hbm.at[pl.ds(e0, EW)], jb)
    pltpu.sync_copy(i_hbm.at[pl.ds(e0, EW)], ib)
    plsc.subcore_barrier()
    SUB = 128

    @pl.loop(0, EW, step=2 * SUB)
    def _chunk(c0):
        cps = [
            pltpu.async_copy(shr.at[ib.at[pl.ds(c0, SUB)]], sa, gsem),
            pltpu.async_copy(shr.at[jb.at[pl.ds(c0, SUB)]], sb, gsem),
            pltpu.async_copy(shr.at[ib.at[pl.ds(c0 + SUB, SUB)]], sa2,
                             gsem),
            pltpu.async_copy(shr.at[jb.at[pl.ds(c0 + SUB, SUB)]], sb2,
                             gsem),
        ]
        for cp in cps:
            cp.wait()
        outs = [
            pltpu.async_copy(sa, s1i_hbm.at[pl.ds(e0 + c0, SUB)], ssem),
            pltpu.async_copy(sb, s1j_hbm.at[pl.ds(e0 + c0, SUB)], ssem),
            pltpu.async_copy(sa2, s1i_hbm.at[pl.ds(e0 + c0 + SUB, SUB)],
                             ssem),
            pltpu.async_copy(sb2, s1j_hbm.at[pl.ds(e0 + c0 + SUB, SUB)],
                             ssem),
        ]
        for cp in outs:
            cp.wait()


def _sgather_kernel(j, i, s1p):
    i32 = jnp.int32
    HD = SDIM // 2
    kern = pl.kernel(
        _sgather_body,
        out_type=(
            jax.ShapeDtypeStruct((E, HD), i32),
            jax.ShapeDtypeStruct((E, HD), i32),
        ),
        mesh=_sc_mesh,
        scratch_types=[
            pltpu.VMEM((EW,), i32),
            pltpu.VMEM((EW,), i32),
            pltpu.VMEM((128, HD), i32),
            pltpu.VMEM((128, HD), i32),
            pltpu.VMEM((128, HD), i32),
            pltpu.VMEM((128, HD), i32),
            pltpu.VMEM_SHARED((N, HD), i32),
            pltpu.SemaphoreType.DMA,
            pltpu.SemaphoreType.DMA,
        ],
        compiler_params=_sc_params(True),
    )
    return kern(j, i, s1p)


def _unpack_bf16_pair(x_i32):
    bf = jnp.bfloat16
    lo = lax.bitcast_convert_type(x_i32.astype(jnp.uint32).astype(jnp.uint16),
                                  bf)
    hi = lax.bitcast_convert_type(
        lax.shift_right_logical(x_i32.astype(jnp.uint32),
                                jnp.uint32(16)).astype(jnp.uint16), bf)
    return lo, hi


def _edge_body(s1ip_ref, s1jp_ref, esym_ref, cdiff_ref, Wb_ref,
               bb_ref, W0a_ref, w0d_ref, b0_ref, W1_ref, b1_ref, out_ref):
    f32 = jnp.float32
    bf = jnp.bfloat16
    HD = SDIM // 2
    ilo, ihi = _unpack_bf16_pair(s1ip_ref[...])
    jlo, jhi = _unpack_bf16_pair(s1jp_ref[...])
    g = (jnp.dot(esym_ref[...].astype(bf), Wb_ref[...],
                 preferred_element_type=f32) + bb_ref[...])
    f_lo = ilo + jlo + g[:, :HD].astype(bf)
    f_hi = ihi + jhi + g[:, HD:].astype(bf)
    cd = cdiff_ref[...]
    d = jnp.dot(cd * cd, jnp.ones((EDGE_DIM, 1), f32),
                preferred_element_type=f32)
    h = jax.nn.silu(
        jnp.dot(f_lo, W0a_ref[:HD, :], preferred_element_type=f32)
        + jnp.dot(f_hi, W0a_ref[HD:, :], preferred_element_type=f32)
        + d * w0d_ref[...] + b0_ref[...])
    out_ref[...] = (
        jnp.dot(h.astype(bf), W1_ref[...], preferred_element_type=f32)
        + b1_ref[...])


def _edge_kernel(s1ip, s1jp, esym, cdiff, W_bond, b_bond, W0a, w0d,
                 b_b0, W_b1, b_b1):
    f32 = jnp.float32
    EB = 8192
    HD = SDIM // 2
    grid = (E // EB,)
    row_spec = lambda w: pl.BlockSpec((EB, w), lambda g: (g, 0))
    full = lambda a, b: pl.BlockSpec((a, b), lambda g: (0, 0))
    return pl.pallas_call(
        _edge_body,
        grid=grid,
        in_specs=[
            row_spec(HD), row_spec(HD), row_spec(EDGE_DIM),
            row_spec(EDGE_DIM),
            full(EDGE_DIM, SDIM), full(1, SDIM),
            full(SDIM, SDIM), full(1, SDIM), full(1, SDIM),
            full(SDIM, 5), full(1, 5),
        ],
        out_specs=pl.BlockSpec((EB, 5), lambda g: (g, 0)),
        out_shape=jax.ShapeDtypeStruct((E, 5), f32),
    )(s1ip, s1jp, esym, cdiff, W_bond.astype(jnp.bfloat16), b_bond,
      W0a.astype(jnp.bfloat16), w0d, b_b0, W_b1.astype(jnp.bfloat16), b_b1)


@jax.jit
def kernel(s, v, p, e, batch, edge_index_global,
           W_shared, b_shared, W_coords, W_atoms, b_atoms,
           W_bond, b_bond, W_b0, b_b0, W_b1, b_b1):
    f32 = jnp.float32
    j = edge_index_global[0]
    i = edge_index_global[1]

    v0 = v[:, 0, :]
    v1 = v[:, 1, :]
    v2 = v[:, 2, :]
    p16 = jnp.pad(p, ((0, 0), (0, 13)))
    s1, atoms_pred, c16 = _node_kernel(
        s, v0, v1, v2, p16, batch[:, None], W_shared, b_shared[None, :],
        W_coords, W_atoms, b_atoms[None, :])
    coords_pred = c16[:, :3]

    key = jnp.bitwise_or(lax.shift_left(j, 10), i)
    rkey = jnp.bitwise_or(lax.shift_left(i, 10), j)
    neg1 = jnp.full((RANGE,), -1, jnp.int32)
    table = _build_table(key, neg1)
    e17 = jnp.concatenate([e, jnp.zeros((NSENT, EDGE_DIM), f32)], axis=0)
    esym, cdiff = _esym_kernel(j, i, key, rkey, table, e17, c16)
    s1b = s1.astype(jnp.bfloat16)
    lo16 = lax.bitcast_convert_type(s1b[:, :SDIM // 2],
                                    jnp.uint16).astype(jnp.uint32)
    hi16 = lax.bitcast_convert_type(s1b[:, SDIM // 2:],
                                    jnp.uint16).astype(jnp.uint32)
    s1p = jnp.bitwise_or(
        lo16, lax.shift_left(hi16, jnp.uint32(16))).astype(jnp.int32)

    W0a = W_b0[:SDIM]
    w0d = W_b0[SDIM:SDIM + 1]
    s1ip, s1jp = _sgather_kernel(j, i, s1p)
    bonds_pred = _edge_kernel(s1ip, s1jp, esym, cdiff, W_bond,
                              b_bond[None, :], W0a, w0d, b_b0[None, :],
                              W_b1, b_b1[None, :])
    return coords_pred, atoms_pred, bonds_pred


# speedup vs baseline: 1.0069x; 1.0069x over previous
import dataclasses
import functools

import jax
import jax.numpy as jnp
from jax import lax
from jax.experimental import pallas as pl
from jax.experimental.pallas import tpu as pltpu
from jax.experimental.pallas import tpu_sc as plsc

N = 1024
E = 32768
SDIM = 256
VDIM = 64
EDGE_DIM = 16
NUM_GRAPHS = 32

NKEYS = N * N
NW = 32
RANGE = NKEYS // NW
EW = E // NW
CHUNK = 128
KEY_CHUNK = 2048
NSENT = 128

_sc_mesh = plsc.VectorSubcoreMesh(core_axis_name="c", subcore_axis_name="s")


def _sc_params(tc_tiling):
    cp = pltpu.CompilerParams()
    fields = pltpu.CompilerParams.__dataclass_fields__
    if "needs_layout_passes" in fields:
        cp = dataclasses.replace(cp, needs_layout_passes=False)
    if "use_tc_tiling_on_sc" in fields:
        cp = dataclasses.replace(cp, use_tc_tiling_on_sc=tc_tiling)
    return cp


def _node_body(s_ref, v0_ref, v1_ref, v2_ref, p16_ref, b_ref, Ws_ref, bs_ref,
               Wc_ref, Wa_ref, ba_ref, s1_ref, atoms_ref, c16_ref):
    f32 = jnp.float32
    s1 = jax.nn.silu(
        jnp.dot(s_ref[...], Ws_ref[...], preferred_element_type=f32)
        + bs_ref[...])
    s1_ref[...] = s1
    atoms_ref[...] = (
        jnp.dot(s1, Wa_ref[...], preferred_element_type=f32) + ba_ref[...])

    Wc = Wc_ref[...]
    c3 = jnp.concatenate(
        [jnp.dot(v0_ref[...], Wc, preferred_element_type=f32),
         jnp.dot(v1_ref[...], Wc, preferred_element_type=f32),
         jnp.dot(v2_ref[...], Wc, preferred_element_type=f32),
         jnp.zeros((N, 13), f32)], axis=1)
    c16 = c3 + p16_ref[...]
    iota = lax.broadcasted_iota(jnp.int32, (N, NUM_GRAPHS), 1)
    oh = (b_ref[...] == iota).astype(f32)
    ones = jnp.ones((N, 1), f32)
    seg = lax.dot_general(oh, c16, (((0,), (0,)), ((), ())),
                          preferred_element_type=f32)
    cnt = lax.dot_general(oh, ones, (((0,), (0,)), ((), ())),
                          preferred_element_type=f32)
    mean = seg / jnp.maximum(cnt, 1.0)
    c16_ref[...] = c16 - jnp.dot(oh, mean, preferred_element_type=f32)


def _node_kernel(s, v0, v1, v2, p16, batch2, W_shared, b_shared, W_coords,
                 W_atoms, b_atoms):
    f32 = jnp.float32
    return pl.pallas_call(
        _node_body,
        out_shape=(
            jax.ShapeDtypeStruct((N, SDIM), f32),
            jax.ShapeDtypeStruct((N, 16), f32),
            jax.ShapeDtypeStruct((N, 16), f32),
        ),
    )(s, v0, v1, v2, p16, batch2, W_shared, b_shared, W_coords, W_atoms,
      b_atoms)


def _table_body(key_hbm, neg1_hbm, table_hbm, tab, kb0, kb1, sem0, sem1):
    wid = lax.axis_index("s") * 2 + lax.axis_index("c")
    pltpu.sync_copy(neg1_hbm, tab)
    iota = lax.broadcasted_iota(jnp.int32, (16,), 0)

    def start_load(c0, kb, sem):
        pltpu.async_copy(key_hbm.at[pl.ds(c0, KEY_CHUNK)], kb, sem)

    def wait_load(kb, sem):
        pltpu.make_async_copy(key_hbm.at[pl.ds(0, KEY_CHUNK)], kb, sem).wait()

    def process(c0, kb):
        @pl.loop(0, KEY_CHUNK, step=128)
        def _vreg(t):
            for u in range(0, 128, 16):
                key = kb[pl.ds(t + u, 16)]
                inr = lax.shift_right_logical(key, 15) == wid
                _, keep = plsc.scan_count(key)
                m = jnp.logical_and(inr, keep)
                idx = jnp.bitwise_and(key, RANGE - 1)
                kvec = c0 + t + u + iota
                plsc.store_scatter(tab, [idx], kvec, mask=m)

    start_load(0, kb0, sem0)

    @pl.loop(0, E, step=2 * KEY_CHUNK)
    def _chunk(c0):
        wait_load(kb0, sem0)

        @pl.when(c0 + KEY_CHUNK < E)
        def _():
            start_load(c0 + KEY_CHUNK, kb1, sem1)

        process(c0, kb0)
        wait_load(kb1, sem1)

        @pl.when(c0 + 2 * KEY_CHUNK < E)
        def _():
            start_load(c0 + 2 * KEY_CHUNK, kb0, sem0)

        process(c0 + KEY_CHUNK, kb1)

    pltpu.sync_copy(tab, table_hbm.at[pl.ds(wid * RANGE, RANGE)])


def _build_table(key, neg1):
    i32 = jnp.int32
    kern = pl.kernel(
        _table_body,
        out_type=jax.ShapeDtypeStruct((NKEYS,), i32),
        mesh=_sc_mesh,
        scratch_types=[
            pltpu.VMEM((RANGE,), i32),
            pltpu.VMEM((KEY_CHUNK,), i32),
            pltpu.VMEM((KEY_CHUNK,), i32),
            pltpu.SemaphoreType.DMA,
            pltpu.SemaphoreType.DMA,
        ],
        compiler_params=_sc_params(False),
    )
    return kern(key, neg1)


def _esym_body(j_hbm, i_hbm, key_hbm, rkey_hbm, tab_hbm, e17_hbm, c16_hbm,
               esym_hbm, cdiff_hbm,
               jb, ib, keyb, rkeyb, fwdb, bwdb, ef, eb, cif, cjf,
               shr_c, csem, tsem, esem, ssem):
    cid = lax.axis_index("c")
    sid = lax.axis_index("s")
    wid = sid * 2 + cid
    e0 = wid * EW

    @pl.when(sid == 0)
    def _():
        pltpu.sync_copy(c16_hbm, shr_c)

    pltpu.sync_copy(j_hbm.at[pl.ds(e0, EW)], jb)
    pltpu.sync_copy(i_hbm.at[pl.ds(e0, EW)], ib)
    pltpu.sync_copy(key_hbm.at[pl.ds(e0, EW)], keyb)
    pltpu.sync_copy(rkey_hbm.at[pl.ds(e0, EW)], rkeyb)
    iota = lax.broadcasted_iota(jnp.int32, (16,), 0)
    plsc.subcore_barrier()

    ccps = []
    for c0 in range(0, EW, CHUNK):
        ccps.append(pltpu.async_copy(
            shr_c.at[ib.at[pl.ds(c0, CHUNK)]],
            cif.at[pl.ds(c0, CHUNK)], csem))
        ccps.append(pltpu.async_copy(
            shr_c.at[jb.at[pl.ds(c0, CHUNK)]],
            cjf.at[pl.ds(c0, CHUNK)], csem))

    tcps = []
    for c0 in range(0, EW, CHUNK):
        tcps.append(pltpu.async_copy(
            tab_hbm.at[keyb.at[pl.ds(c0, CHUNK)]],
            fwdb.at[pl.ds(c0, CHUNK)], tsem))
        tcps.append(pltpu.async_copy(
            tab_hbm.at[rkeyb.at[pl.ds(c0, CHUNK)]],
            bwdb.at[pl.ds(c0, CHUNK)], tsem))
    for cp in tcps:
        cp.wait()

    @pl.loop(0, EW, step=16)
    def _fix(t):
        b = bwdb[pl.ds(t, 16)]
        sent = E + jnp.bitwise_and(t + iota, NSENT - 1)
        bwdb[pl.ds(t, 16)] = jnp.where(b < 0, sent, b)

    ecps = []
    for c0 in range(0, EW, CHUNK):
        ecps.append(pltpu.async_copy(
            e17_hbm.at[fwdb.at[pl.ds(c0, CHUNK)]],
            ef.at[pl.ds(c0, CHUNK)], esem))
        ecps.append(pltpu.async_copy(
            e17_hbm.at[bwdb.at[pl.ds(c0, CHUNK)]],
            eb.at[pl.ds(c0, CHUNK)], esem))
    for cp in ecps:
        cp.wait()
    for cp in ccps:
        cp.wait()

    @pl.loop(0, EW, step=4)
    def _row(r):
        for u in range(4):
            ef[r + u] = (ef[r + u] + eb[r + u]) * 0.5
            cif[r + u] = cif[r + u] - cjf[r + u]

    pltpu.async_copy(ef, esym_hbm.at[pl.ds(e0, EW)], ssem)
    pltpu.async_copy(cif, cdiff_hbm.at[pl.ds(e0, EW)], ssem)
    pltpu.make_async_copy(ef, esym_hbm.at[pl.ds(e0, EW)], ssem).wait()
    pltpu.make_async_copy(cif, cdiff_hbm.at[pl.ds(e0, EW)], ssem).wait()


def _esym_kernel(j, i, key, rkey, table, e17, c16):
    f32 = jnp.float32
    i32 = jnp.int32
    kern = pl.kernel(
        _esym_body,
        out_type=(
            jax.ShapeDtypeStruct((E, EDGE_DIM), f32),
            jax.ShapeDtypeStruct((E, EDGE_DIM), f32),
        ),
        mesh=_sc_mesh,
        scratch_types=[
            pltpu.VMEM((EW,), i32),
            pltpu.VMEM((EW,), i32),
            pltpu.VMEM((EW,), i32),
            pltpu.VMEM((EW,), i32),
            pltpu.VMEM((EW,), i32),
            pltpu.VMEM((EW,), i32),
            pltpu.VMEM((EW, EDGE_DIM), f32),
            pltpu.VMEM((EW, EDGE_DIM), f32),
            pltpu.VMEM((EW, EDGE_DIM), f32),
            pltpu.VMEM((EW, EDGE_DIM), f32),
            pltpu.VMEM_SHARED((N, EDGE_DIM), f32),
            pltpu.SemaphoreType.DMA,
            pltpu.SemaphoreType.DMA,
            pltpu.SemaphoreType.DMA,
            pltpu.SemaphoreType.DMA,
        ],
        compiler_params=_sc_params(False),
    )
    return kern(j, i, key, rkey, table, e17, c16)


def _sgather_body(j_hbm, i_hbm, s1_hbm, s1i_hbm, s1j_hbm,
                  jb, ib, sa, sb, sa2, sb2, shr, gsem, ssem):
    cid = lax.axis_index("c")
    sid = lax.axis_index("s")
    wid = sid * 2 + cid
    e0 = wid * EW

    @pl.when(sid == 0)
    def _():
        pltpu.sync_copy(s1_hbm, shr)

    pltpu.sync_copy(j_hbm.at[pl.ds(e0, EW)], jb)
    pltpu.sync_copy(i_hbm.at[pl.ds(e0, EW)], ib)
    plsc.subcore_barrier()
    SUB = 128

    @pl.loop(0, EW, step=2 * SUB)
    def _chunk(c0):
        cps = [
            pltpu.async_copy(shr.at[ib.at[pl.ds(c0, SUB)]], sa, gsem),
            pltpu.async_copy(shr.at[jb.at[pl.ds(c0, SUB)]], sb, gsem),
            pltpu.async_copy(shr.at[ib.at[pl.ds(c0 + SUB, SUB)]], sa2,
                             gsem),
            pltpu.async_copy(shr.at[jb.at[pl.ds(c0 + SUB, SUB)]], sb2,
                             gsem),
        ]
        for cp in cps:
            cp.wait()
        outs = [
            pltpu.async_copy(sa, s1i_hbm.at[pl.ds(e0 + c0, SUB)], ssem),
            pltpu.async_copy(sb, s1j_hbm.at[pl.ds(e0 + c0, SUB)], ssem),
            pltpu.async_copy(sa2, s1i_hbm.at[pl.ds(e0 + c0 + SUB, SUB)],
                             ssem),
            pltpu.async_copy(sb2, s1j_hbm.at[pl.ds(e0 + c0 + SUB, SUB)],
                             ssem),
        ]
        for cp in outs:
            cp.wait()


def _sgather_kernel(j, i, s1p):
    i32 = jnp.int32
    HD = SDIM // 2
    kern = pl.kernel(
        _sgather_body,
        out_type=(
            jax.ShapeDtypeStruct((E, HD), i32),
            jax.ShapeDtypeStruct((E, HD), i32),
        ),
        mesh=_sc_mesh,
        scratch_types=[
            pltpu.VMEM((EW,), i32),
            pltpu.VMEM((EW,), i32),
            pltpu.VMEM((128, HD), i32),
            pltpu.VMEM((128, HD), i32),
            pltpu.VMEM((128, HD), i32),
            pltpu.VMEM((128, HD), i32),
            pltpu.VMEM_SHARED((N, HD), i32),
            pltpu.SemaphoreType.DMA,
            pltpu.SemaphoreType.DMA,
        ],
        compiler_params=_sc_params(True),
    )
    return kern(j, i, s1p)


def _unpack_bf16_pair(x_i32):
    bf = jnp.bfloat16
    lo = lax.bitcast_convert_type(x_i32.astype(jnp.uint32).astype(jnp.uint16),
                                  bf)
    hi = lax.bitcast_convert_type(
        lax.shift_right_logical(x_i32.astype(jnp.uint32),
                                jnp.uint32(16)).astype(jnp.uint16), bf)
    return lo, hi


def _edge_body(s1ip_ref, s1jp_ref, esym_ref, cdiff_ref, Wb_ref,
               bb_ref, W0a_ref, w0d_ref, b0_ref, W1_ref, b1_ref, out_ref):
    f32 = jnp.float32
    bf = jnp.bfloat16
    HD = SDIM // 2
    ilo, ihi = _unpack_bf16_pair(s1ip_ref[...])
    jlo, jhi = _unpack_bf16_pair(s1jp_ref[...])
    g = (jnp.dot(esym_ref[...].astype(bf), Wb_ref[...],
                 preferred_element_type=f32) + bb_ref[...])
    f_lo = ilo + jlo + g[:, :HD].astype(bf)
    f_hi = ihi + jhi + g[:, HD:].astype(bf)
    cd = cdiff_ref[...]
    d = jnp.dot(cd * cd, jnp.ones((EDGE_DIM, 1), f32),
                preferred_element_type=f32)
    h = jax.nn.silu(
        jnp.dot(f_lo, W0a_ref[:HD, :], preferred_element_type=f32)
        + jnp.dot(f_hi, W0a_ref[HD:, :], preferred_element_type=f32)
        + d * w0d_ref[...] + b0_ref[...])
    out_ref[...] = (
        jnp.dot(h.astype(bf), W1_ref[...], preferred_element_type=f32)
        + b1_ref[...])


def _edge_kernel(s1ip, s1jp, esym, cdiff, W_bond, b_bond, W0a, w0d,
                 b_b0, W_b1, b_b1):
    f32 = jnp.float32
    EB = 4096
    HD = SDIM // 2
    grid = (E // EB,)
    row_spec = lambda w: pl.BlockSpec((EB, w), lambda g: (g, 0))
    full = lambda a, b: pl.BlockSpec((a, b), lambda g: (0, 0))
    return pl.pallas_call(
        _edge_body,
        grid=grid,
        in_specs=[
            row_spec(HD), row_spec(HD), row_spec(EDGE_DIM),
            row_spec(EDGE_DIM),
            full(EDGE_DIM, SDIM), full(1, SDIM),
            full(SDIM, SDIM), full(1, SDIM), full(1, SDIM),
            full(SDIM, 5), full(1, 5),
        ],
        out_specs=pl.BlockSpec((EB, 5), lambda g: (g, 0)),
        out_shape=jax.ShapeDtypeStruct((E, 5), f32),
    )(s1ip, s1jp, esym, cdiff, W_bond.astype(jnp.bfloat16), b_bond,
      W0a.astype(jnp.bfloat16), w0d, b_b0, W_b1.astype(jnp.bfloat16), b_b1)


@jax.jit
def kernel(s, v, p, e, batch, edge_index_global,
           W_shared, b_shared, W_coords, W_atoms, b_atoms,
           W_bond, b_bond, W_b0, b_b0, W_b1, b_b1):
    f32 = jnp.float32
    j = edge_index_global[0]
    i = edge_index_global[1]

    v0 = v[:, 0, :]
    v1 = v[:, 1, :]
    v2 = v[:, 2, :]
    p16 = jnp.pad(p, ((0, 0), (0, 13)))
    s1, atoms_pred, c16 = _node_kernel(
        s, v0, v1, v2, p16, batch[:, None], W_shared, b_shared[None, :],
        W_coords, W_atoms, b_atoms[None, :])
    coords_pred = c16[:, :3]

    key = jnp.bitwise_or(lax.shift_left(j, 10), i)
    rkey = jnp.bitwise_or(lax.shift_left(i, 10), j)
    neg1 = jnp.full((RANGE,), -1, jnp.int32)
    table = _build_table(key, neg1)
    e17 = jnp.concatenate([e, jnp.zeros((NSENT, EDGE_DIM), f32)], axis=0)
    esym, cdiff = _esym_kernel(j, i, key, rkey, table, e17, c16)
    s1b = s1.astype(jnp.bfloat16)
    lo16 = lax.bitcast_convert_type(s1b[:, :SDIM // 2],
                                    jnp.uint16).astype(jnp.uint32)
    hi16 = lax.bitcast_convert_type(s1b[:, SDIM // 2:],
                                    jnp.uint16).astype(jnp.uint32)
    s1p = jnp.bitwise_or(
        lo16, lax.shift_left(hi16, jnp.uint32(16))).astype(jnp.int32)

    W0a = W_b0[:SDIM]
    w0d = W_b0[SDIM:SDIM + 1]
    s1ip, s1jp = _sgather_kernel(j, i, s1p)
    bonds_pred = _edge_kernel(s1ip, s1jp, esym, cdiff, W_bond,
                              b_bond[None, :], W0a, w0d, b_b0[None, :],
                              W_b1, b_b1[None, :])
    return coords_pred, atoms_pred, bonds_pred
